# Initial kernel scaffold; baseline (speedup 1.0000x reference)
import jax
import jax.numpy as jnp
from jax.experimental import pallas as pl


def kernel(messages, logits):
    raise NotImplementedError("write your pallas kernel here")


# speedup vs baseline: 241.0814x; 241.0814x over previous
import functools

import numpy as np
import jax
import jax.numpy as jnp
from jax import lax
from jax.experimental import pallas as pl
from jax.experimental.pallas import tpu as pltpu
from jax.experimental.pallas import tpu_sc as plsc

B, L, V = 1024, 50, 64
P = 0.05
N = B * L
NC, NS = 2, 16
NW = NC * NS
ROWS_W = N // NW
CH = 200
STEPS = ROWS_W // CH
CW = CH * V
PAD = 16
SCALE_S = 1.0 / (V - 2)
SCALE_G = float(V - 1) / (V - 2)
PR = float(P / (V - 2))


def _shifted_mask() -> np.ndarray:
    mask = np.random.RandomState(42).rand(N, V - 1) < P
    a = np.zeros((N, V), np.float32)
    a[:, : V - 1] = mask
    ash = np.empty(N * V, np.float32)
    ash[1:] = a.reshape(-1)[:-1]
    ash[0] = 0.0
    return ash


_ASH = _shifted_mask()


def _sc_messages(m_flat, ash_flat):
    mesh = plsc.VectorSubcoreMesh(core_axis_name="c", subcore_axis_name="s")

    @functools.partial(
        pl.kernel,
        out_type=jax.ShapeDtypeStruct((N * V,), jnp.float32),
        mesh=mesh,
        scratch_types=[
            pltpu.VMEM((PAD + CW,), jnp.float32),
            pltpu.VMEM((CW,), jnp.float32),
            pltpu.VMEM((CW,), jnp.float32),
            pltpu.SemaphoreType.DMA,
        ],
        compiler_params=pltpu.CompilerParams(needs_layout_passes=False),
    )
    def k(m_hbm, ash_hbm, out_hbm, mbuf, abuf, obuf, sem):
        wid = lax.axis_index("s") * NC + lax.axis_index("c")
        base_w = wid * (ROWS_W * V)
        for step in range(STEPS):
            base = base_w + step * CW
            cm = pltpu.async_copy(m_hbm.at[pl.ds(base, CW)], mbuf.at[pl.ds(PAD, CW)], sem)
            ca = pltpu.async_copy(ash_hbm.at[pl.ds(base, CW)], abuf, sem)
            cm.wait()
            ca.wait()

            def row_body(r, carry):
                rb = r * V
                gs = []
                for kk in range(4):
                    mp = mbuf[pl.ds(PAD - 1 + rb + kk * 16, 16)]
                    av = abuf[pl.ds(rb + kk * 16, 16)]
                    gs.append(mp * av)
                s = jnp.sum(gs[0] + gs[1] + gs[2] + gs[3]) * SCALE_S
                sv = jnp.full((16,), s, jnp.float32)
                sv0 = jnp.where(lax.iota(jnp.int32, 16) > 0, sv, 0.0)
                for kk in range(4):
                    mm = mbuf[pl.ds(PAD + rb + kk * 16, 16)]
                    add = sv0 if kk == 0 else sv
                    obuf[pl.ds(rb + kk * 16, 16)] = mm + add - SCALE_G * gs[kk]
                return carry

            lax.fori_loop(0, CH, row_body, 0)
            pltpu.sync_copy(obuf, out_hbm.at[pl.ds(base, CW)])

    return k(m_flat, ash_flat)


def _tc_logits(l2d):
    BR = 800

    def body(l_ref, o_ref):
        l = l_ref[...]
        e = jnp.exp(l)
        e0 = e[:, 0:1]
        q = (1.0 - P) * e + PR * jnp.clip(1.0 - e - e0, 0.0, 1.0)
        col = lax.broadcasted_iota(jnp.int32, l.shape, 1)
        o_ref[...] = jnp.where(col == 0, l, jnp.log(q))

    return pl.pallas_call(
        body,
        grid=(N // BR,),
        in_specs=[pl.BlockSpec((BR, V), lambda i: (i, 0))],
        out_specs=pl.BlockSpec((BR, V), lambda i: (i, 0)),
        out_shape=jax.ShapeDtypeStruct((N, V), jnp.float32),
    )(l2d)


def kernel(messages, logits):
    m_flat = messages.reshape(N * V)
    mn = _sc_messages(m_flat, jnp.asarray(_ASH)).reshape(B, L, V)
    ln = _tc_logits(logits.reshape(N, V)).reshape(B, L, V)
    return (mn, ln, messages, logits)


# speedup vs baseline: 273.1655x; 1.1331x over previous
import functools

import numpy as np
import jax
import jax.numpy as jnp
from jax import lax
from jax.experimental import pallas as pl
from jax.experimental.pallas import tpu as pltpu
from jax.experimental.pallas import tpu_sc as plsc

B, L, V = 1024, 50, 64
P = 0.05
N = B * L
NC, NS = 2, 16
NW = NC * NS
ROWS_W = N // NW
CH = 200
STEPS = ROWS_W // CH
CW = CH * V
PAD = 16
SCALE_S = 1.0 / (V - 2)
SCALE_G = float(V - 1) / (V - 2)
PR = float(P / (V - 2))


def _shifted_mask() -> np.ndarray:
    mask = np.random.RandomState(42).rand(N, V - 1) < P
    a = np.zeros((N, V), np.float32)
    a[:, : V - 1] = mask
    ash = np.empty(N * V, np.float32)
    ash[1:] = a.reshape(-1)[:-1]
    ash[0] = 0.0
    return ash


_ASH = _shifted_mask()


def _sc_messages(m_flat, ash_flat):
    mesh = plsc.VectorSubcoreMesh(core_axis_name="c", subcore_axis_name="s")

    @functools.partial(
        pl.kernel,
        out_type=jax.ShapeDtypeStruct((N * V,), jnp.float32),
        mesh=mesh,
        scratch_types=[
            pltpu.VMEM((PAD + CW,), jnp.float32),
            pltpu.VMEM((CW,), jnp.float32),
            pltpu.VMEM((CW,), jnp.float32),
            pltpu.SemaphoreType.DMA,
        ],
        compiler_params=pltpu.CompilerParams(needs_layout_passes=False),
    )
    def k(m_hbm, ash_hbm, out_hbm, mbuf, abuf, obuf, sem):
        wid = lax.axis_index("s") * NC + lax.axis_index("c")
        base_w = wid * (ROWS_W * V)
        for step in range(STEPS):
            base = base_w + step * CW
            cm = pltpu.async_copy(m_hbm.at[pl.ds(base, CW)], mbuf.at[pl.ds(PAD, CW)], sem)
            ca = pltpu.async_copy(ash_hbm.at[pl.ds(base, CW)], abuf, sem)
            cm.wait()
            ca.wait()

            def row_body(r, carry):
                rb = r * V
                gs = []
                for kk in range(4):
                    mp = mbuf[pl.ds(PAD - 1 + rb + kk * 16, 16)]
                    av = abuf[pl.ds(rb + kk * 16, 16)]
                    gs.append(mp * av)
                s = jnp.sum(gs[0] + gs[1] + gs[2] + gs[3]) * SCALE_S
                sv = jnp.full((16,), s, jnp.float32)
                sv0 = jnp.where(lax.iota(jnp.int32, 16) > 0, sv, 0.0)
                for kk in range(4):
                    mm = mbuf[pl.ds(PAD + rb + kk * 16, 16)]
                    add = sv0 if kk == 0 else sv
                    obuf[pl.ds(rb + kk * 16, 16)] = mm + add - SCALE_G * gs[kk]
                return carry

            lax.fori_loop(0, CH, row_body, 0)
            pltpu.sync_copy(obuf, out_hbm.at[pl.ds(base, CW)])

    return k(m_flat, ash_flat)


def _tc_logits(l3d):
    BB = 64

    def body(l_ref, o_ref):
        l = l_ref[...]
        e = jnp.exp(l)
        e0 = e[:, :, 0:1]
        q = (1.0 - P) * e + PR * jnp.clip(1.0 - e - e0, 0.0, 1.0)
        col = lax.broadcasted_iota(jnp.int32, l.shape, 2)
        o_ref[...] = jnp.where(col == 0, l, jnp.log(q))

    return pl.pallas_call(
        body,
        grid=(B // BB,),
        in_specs=[pl.BlockSpec((BB, L, V), lambda i: (i, 0, 0))],
        out_specs=pl.BlockSpec((BB, L, V), lambda i: (i, 0, 0)),
        out_shape=jax.ShapeDtypeStruct((B, L, V), jnp.float32),
    )(l3d)


def kernel(messages, logits):
    m_flat = messages.reshape(N * V)
    mn = _sc_messages(m_flat, jnp.asarray(_ASH)).reshape(B, L, V)
    ln = _tc_logits(logits)
    return (mn, ln, messages, logits)


# speedup vs baseline: 273.7359x; 1.0021x over previous
import functools

import numpy as np
import jax
import jax.numpy as jnp
from jax import lax
from jax.experimental import pallas as pl
from jax.experimental.pallas import tpu as pltpu
from jax.experimental.pallas import tpu_sc as plsc

B, L, V = 1024, 50, 64
P = 0.05
N = B * L
NC, NS = 2, 16
NW = NC * NS
ROWS_W = N // NW
CH = 200
STEPS = ROWS_W // CH
CW = CH * V
PAD = 16
SCALE_S = 1.0 / (V - 2)
SCALE_G = float(V - 1) / (V - 2)
PR = float(P / (V - 2))


def _shifted_mask() -> np.ndarray:
    mask = np.random.RandomState(42).rand(N, V - 1) < P
    a = np.zeros((N, V), np.float32)
    a[:, : V - 1] = mask
    ash = np.empty(N * V, np.float32)
    ash[1:] = a.reshape(-1)[:-1]
    ash[0] = 0.0
    return ash


_ASH = _shifted_mask()


def _sc_messages(m_flat, ash_flat):
    mesh = plsc.VectorSubcoreMesh(core_axis_name="c", subcore_axis_name="s")

    @functools.partial(
        pl.kernel,
        out_type=jax.ShapeDtypeStruct((N * V,), jnp.float32),
        mesh=mesh,
        scratch_types=[
            pltpu.VMEM((PAD + CW,), jnp.float32),
            pltpu.VMEM((CW,), jnp.float32),
            pltpu.VMEM((CW,), jnp.float32),
            pltpu.SemaphoreType.DMA,
        ],
        compiler_params=pltpu.CompilerParams(needs_layout_passes=False),
    )
    def k(m_hbm, ash_hbm, out_hbm, mbuf, abuf, obuf, sem):
        wid = lax.axis_index("s") * NC + lax.axis_index("c")
        base_w = wid * (ROWS_W * V)
        for step in range(STEPS):
            base = base_w + step * CW
            cm = pltpu.async_copy(m_hbm.at[pl.ds(base, CW)], mbuf.at[pl.ds(PAD, CW)], sem)
            ca = pltpu.async_copy(ash_hbm.at[pl.ds(base, CW)], abuf, sem)
            cm.wait()
            ca.wait()

            def row_body(r, carry):
                rb = r * V
                gs = []
                for kk in range(4):
                    mp = mbuf[pl.ds(PAD - 1 + rb + kk * 16, 16)]
                    av = abuf[pl.ds(rb + kk * 16, 16)]
                    gs.append(mp * av)
                s = jnp.sum(gs[0] + gs[1] + gs[2] + gs[3]) * SCALE_S
                sv = jnp.full((16,), s, jnp.float32)
                sv0 = jnp.where(lax.iota(jnp.int32, 16) > 0, sv, 0.0)
                for kk in range(4):
                    mm = mbuf[pl.ds(PAD + rb + kk * 16, 16)]
                    add = sv0 if kk == 0 else sv
                    obuf[pl.ds(rb + kk * 16, 16)] = mm + add - SCALE_G * gs[kk]
                return carry

            lax.fori_loop(0, CH, row_body, 0)
            pltpu.sync_copy(obuf, out_hbm.at[pl.ds(base, CW)])

    return k(m_flat, ash_flat)


def _tc_logits(l3d):
    BB = 64

    def body(l_ref, o_ref):
        l = l_ref[...]
        e = jnp.exp(l)
        e0 = e[:, :, 0:1]
        q = (1.0 - P) * e + PR * jnp.clip(1.0 - e - e0, 0.0, 1.0)
        col = lax.broadcasted_iota(jnp.int32, l.shape, 2)
        o_ref[...] = jnp.where(col == 0, l, jnp.log(q))

    return pl.pallas_call(
        body,
        grid=(B // BB,),
        in_specs=[pl.BlockSpec((BB, L, V), lambda i: (i, 0, 0))],
        out_specs=pl.BlockSpec((BB, L, V), lambda i: (i, 0, 0)),
        out_shape=jax.ShapeDtypeStruct((B, L, V), jnp.float32),
    )(l3d)


def kernel(messages, logits):
    ln = _tc_logits(logits)
    m_flat = messages.reshape(N * V)
    mn = _sc_messages(m_flat, jnp.asarray(_ASH)).reshape(B, L, V)
    return (mn, ln, messages, logits)
